# baseline (speedup 1.0000x reference)
import functools

import jax
import jax.numpy as jnp
from jax import lax
from jax.experimental import pallas as pl
from jax.experimental.pallas import tpu as pltpu
from jax.experimental.pallas import tpu_sc as plsc

_COMMITMENT_COST = 0.25

_TN = 4096
_TK = 1024

_H = 2
_NC = 2
_NS = 16
_NW = _NC * _NS


def _vq_argmin_body(x_ref, w_ref, idx_ref, min_ref, loss_ref, xs_ref, x2_ref, ws_ref):
    i = pl.program_id(0)
    k = pl.program_id(1)
    nk = pl.num_programs(1)

    @pl.when(k == 0)
    def _():
        x = x_ref[...]
        xs_ref[...] = jnp.sum(x * x, axis=1, keepdims=True)
        x2_ref[...] = x + x

    @pl.when((i == 0) & (k == 0))
    def _():
        loss_ref[...] = jnp.zeros_like(loss_ref)

    w = w_ref[...]
    tn, tk = x_ref.shape[0], w.shape[0]

    @pl.when(i == 0)
    def _():
        ones = jnp.ones((1, w.shape[1]), jnp.float32)
        ws_ref[0, pl.ds(k * tk, tk)] = lax.dot_general(
            ones, w * w, (((1,), (1,)), ((), ())),
            precision=lax.Precision.HIGHEST,
            preferred_element_type=jnp.float32,
        )[0]

    hk = tk // _H
    xs = xs_ref[...]
    x2 = x2_ref[...]
    lmin = None
    for h in range(_H):
        wh = w_ref[h * hk:(h + 1) * hk, :]
        m2 = lax.dot_general(
            x2, wh, (((1,), (1,)), ((), ())),
            preferred_element_type=jnp.float32,
        )
        wsh = ws_ref[0:1, pl.ds(k * tk + h * hk, hk)]
        d = (xs + wsh) - m2
        hmin = jnp.min(d, axis=1, keepdims=True)
        jloc = lax.broadcasted_iota(jnp.int32, (tn, hk), 1).astype(jnp.float32)
        deq = lax.bitcast_convert_type(d, jnp.int32) == (
            lax.bitcast_convert_type(jnp.broadcast_to(hmin, d.shape),
                                     jnp.int32))
        cand = jnp.where(deq, jloc, jnp.float32(2.0 ** 30))
        hidx = (jnp.min(cand, axis=1, keepdims=True).astype(jnp.int32)
                + (k * tk + h * hk))
        if lmin is None:
            lmin, lidx = hmin, hidx
        else:
            hbetter = hmin < lmin
            lidx = jnp.where(hbetter, hidx, lidx)
            lmin = jnp.where(hbetter, hmin, lmin)

    @pl.when(k == 0)
    def _():
        min_ref[...] = lmin
        idx_ref[...] = lidx

    @pl.when(k > 0)
    def _():
        better = lmin < min_ref[...]
        idx_ref[...] = jnp.where(better, lidx, idx_ref[...])
        min_ref[...] = jnp.where(better, lmin, min_ref[...])

    @pl.when(k == nk - 1)
    def _():
        loss_ref[...] += jnp.sum(min_ref[...], keepdims=True)


def _vq_argmin(inputs, W):
    n, d = inputs.shape
    kk = W.shape[0]
    grid = (n // _TN, kk // _TK)
    return pl.pallas_call(
        _vq_argmin_body,
        grid=grid,
        in_specs=[
            pl.BlockSpec((_TN, d), lambda i, k: (i, 0)),
            pl.BlockSpec((_TK, d), lambda i, k: (k, 0)),
        ],
        out_specs=[
            pl.BlockSpec((_TN, 1), lambda i, k: (i, 0)),
            pl.BlockSpec((_TN, 1), lambda i, k: (i, 0)),
            pl.BlockSpec((1, 1), lambda i, k: (0, 0)),
        ],
        out_shape=[
            jax.ShapeDtypeStruct((n, 1), jnp.int32),
            jax.ShapeDtypeStruct((n, 1), jnp.float32),
            jax.ShapeDtypeStruct((1, 1), jnp.float32),
        ],
        scratch_shapes=[
            pltpu.VMEM((_TN, 1), jnp.float32),
            pltpu.VMEM((_TN, d), jnp.float32),
            pltpu.VMEM((1, kk), jnp.float32),
        ],
        compiler_params=pltpu.CompilerParams(
            dimension_semantics=("arbitrary", "arbitrary"),
        ),
    )(inputs, W)


def _gather_body(table_hbm, idx_hbm, out_hbm, idx_v, rows_v, sem):
    b_per_w = idx_v.shape[0]
    chunk = rows_v.shape[0]
    wid = lax.axis_index("s") * _NC + lax.axis_index("c")
    base = wid * b_per_w
    pltpu.sync_copy(idx_hbm.at[pl.ds(base, b_per_w)], idx_v)
    for c in range(b_per_w // chunk):
        pltpu.async_copy(
            table_hbm.at[idx_v.at[pl.ds(c * chunk, chunk)]], rows_v, sem
        ).wait()
        pltpu.sync_copy(rows_v, out_hbm.at[pl.ds(base + c * chunk, chunk)])


def _sc_gather(W, idx_flat):
    kk, d = W.shape
    n = idx_flat.shape[0]
    b_per_w = n // _NW
    chunk = 256
    mesh = plsc.VectorSubcoreMesh(core_axis_name="c", subcore_axis_name="s")
    k = functools.partial(
        pl.kernel,
        mesh=mesh,
        out_type=jax.ShapeDtypeStruct((n, d), jnp.float32),
        scratch_types=[
            pltpu.VMEM((b_per_w,), jnp.int32),
            pltpu.VMEM((chunk, d), jnp.float32),
            pltpu.SemaphoreType.DMA,
        ],
    )(_gather_body)
    return k(W, idx_flat)


def kernel(inputs, W):
    n, d = inputs.shape
    idx, min_d, loss_sum = _vq_argmin(inputs, W)
    quantized = _sc_gather(W, idx.reshape(n))
    vq_loss = loss_sum.reshape(()) * ((1.0 + _COMMITMENT_COST) / (n * d))
    return (quantized, vq_loss, idx)


# speedup vs baseline: 1.1401x; 1.1401x over previous
import functools

import jax
import jax.numpy as jnp
from jax import lax
from jax.experimental import pallas as pl
from jax.experimental.pallas import tpu as pltpu
from jax.experimental.pallas import tpu_sc as plsc

_COMMITMENT_COST = 0.25

_TN = 4096
_TK = 4096

_H = 4
_NC = 2
_NS = 16
_NW = _NC * _NS


def _vq_argmin_body(x_ref, w_ref, idx_ref, min_ref, loss_ref, xs_ref, x2_ref, ws_ref):
    i = pl.program_id(0)
    k = pl.program_id(1)
    nk = pl.num_programs(1)

    @pl.when(k == 0)
    def _():
        x = x_ref[...]
        xs_ref[...] = jnp.sum(x * x, axis=1, keepdims=True)
        x2_ref[...] = x + x

    @pl.when((i == 0) & (k == 0))
    def _():
        loss_ref[...] = jnp.zeros_like(loss_ref)

    w = w_ref[...]
    tn, tk = x_ref.shape[0], w.shape[0]

    @pl.when(i == 0)
    def _():
        ones = jnp.ones((1, w.shape[1]), jnp.float32)
        ws_ref[0, pl.ds(k * tk, tk)] = lax.dot_general(
            ones, w * w, (((1,), (1,)), ((), ())),
            precision=lax.Precision.HIGHEST,
            preferred_element_type=jnp.float32,
        )[0]

    hk = tk // _H
    xs = xs_ref[...]
    x2 = x2_ref[...]
    lmin = None
    for h in range(_H):
        wh = w_ref[h * hk:(h + 1) * hk, :]
        m2 = lax.dot_general(
            x2, wh, (((1,), (1,)), ((), ())),
            preferred_element_type=jnp.float32,
        )
        wsh = ws_ref[0:1, pl.ds(k * tk + h * hk, hk)]
        d = (xs + wsh) - m2
        hmin = jnp.min(d, axis=1, keepdims=True)
        jloc = lax.broadcasted_iota(jnp.int32, (tn, hk), 1).astype(jnp.float32)
        deq = lax.bitcast_convert_type(d, jnp.int32) == (
            lax.bitcast_convert_type(jnp.broadcast_to(hmin, d.shape),
                                     jnp.int32))
        cand = jnp.where(deq, jloc, jnp.float32(2.0 ** 30))
        hidx = (jnp.min(cand, axis=1, keepdims=True).astype(jnp.int32)
                + (k * tk + h * hk))
        if lmin is None:
            lmin, lidx = hmin, hidx
        else:
            hbetter = hmin < lmin
            lidx = jnp.where(hbetter, hidx, lidx)
            lmin = jnp.where(hbetter, hmin, lmin)

    @pl.when(k == 0)
    def _():
        min_ref[...] = lmin
        idx_ref[...] = lidx

    @pl.when(k > 0)
    def _():
        better = lmin < min_ref[...]
        idx_ref[...] = jnp.where(better, lidx, idx_ref[...])
        min_ref[...] = jnp.where(better, lmin, min_ref[...])

    @pl.when(k == nk - 1)
    def _():
        loss_ref[...] += jnp.sum(min_ref[...], keepdims=True)


def _vq_argmin(inputs, W):
    n, d = inputs.shape
    kk = W.shape[0]
    grid = (n // _TN, kk // _TK)
    return pl.pallas_call(
        _vq_argmin_body,
        grid=grid,
        in_specs=[
            pl.BlockSpec((_TN, d), lambda i, k: (i, 0)),
            pl.BlockSpec((_TK, d), lambda i, k: (k, 0)),
        ],
        out_specs=[
            pl.BlockSpec((_TN, 1), lambda i, k: (i, 0)),
            pl.BlockSpec((_TN, 1), lambda i, k: (i, 0)),
            pl.BlockSpec((1, 1), lambda i, k: (0, 0)),
        ],
        out_shape=[
            jax.ShapeDtypeStruct((n, 1), jnp.int32),
            jax.ShapeDtypeStruct((n, 1), jnp.float32),
            jax.ShapeDtypeStruct((1, 1), jnp.float32),
        ],
        scratch_shapes=[
            pltpu.VMEM((_TN, 1), jnp.float32),
            pltpu.VMEM((_TN, d), jnp.float32),
            pltpu.VMEM((1, kk), jnp.float32),
        ],
        compiler_params=pltpu.CompilerParams(
            dimension_semantics=("arbitrary", "arbitrary"),
        ),
    )(inputs, W)


def _gather_body(table_hbm, idx_hbm, out_hbm, idx_v, rows_v, sem):
    b_per_w = idx_v.shape[0]
    chunk = rows_v.shape[0]
    wid = lax.axis_index("s") * _NC + lax.axis_index("c")
    base = wid * b_per_w
    pltpu.sync_copy(idx_hbm.at[pl.ds(base, b_per_w)], idx_v)
    for c in range(b_per_w // chunk):
        pltpu.async_copy(
            table_hbm.at[idx_v.at[pl.ds(c * chunk, chunk)]], rows_v, sem
        ).wait()
        pltpu.sync_copy(rows_v, out_hbm.at[pl.ds(base + c * chunk, chunk)])


def _sc_gather(W, idx_flat):
    kk, d = W.shape
    n = idx_flat.shape[0]
    b_per_w = n // _NW
    chunk = 256
    mesh = plsc.VectorSubcoreMesh(core_axis_name="c", subcore_axis_name="s")
    k = functools.partial(
        pl.kernel,
        mesh=mesh,
        out_type=jax.ShapeDtypeStruct((n, d), jnp.float32),
        scratch_types=[
            pltpu.VMEM((b_per_w,), jnp.int32),
            pltpu.VMEM((chunk, d), jnp.float32),
            pltpu.SemaphoreType.DMA,
        ],
    )(_gather_body)
    return k(W, idx_flat)


def kernel(inputs, W):
    n, d = inputs.shape
    idx, min_d, loss_sum = _vq_argmin(inputs, W)
    quantized = _sc_gather(W, idx.reshape(n))
    vq_loss = loss_sum.reshape(()) * ((1.0 + _COMMITMENT_COST) / (n * d))
    return (quantized, vq_loss, idx)
